# baseline (speedup 1.0000x reference)
import functools

import jax
import jax.numpy as jnp
from jax import lax
from jax.experimental import pallas as pl
from jax.experimental.pallas import tpu as pltpu
from jax.experimental.pallas import tpu_sc as plsc

_NRELS = 100000
_B = 16384
_L = 200
_C = 16

_NC = 2
_NS = 16
_NW = _NC * _NS
_N = _B * _L
_NPW = _N // _NW
_K = 2048
_NCHUNK = _NPW // _K


def _sc_gather_body(
    tbl_hbm, idx_hbm, out_hbm, tbl_v,
    idx_v0, idx_v1, out_v0, out_v1,
    sem_t, si0, si1, so0, so1,
):
    wid = lax.axis_index("s") * _NC + lax.axis_index("c")
    base = wid * _NPW
    idx_bufs = (idx_v0, idx_v1)
    out_bufs = (out_v0, out_v1)
    sin = (si0, si1)
    sout = (so0, so1)

    tcp = pltpu.make_async_copy(tbl_hbm, tbl_v, sem_t)
    tcp.start()
    for par in range(2):
        pltpu.make_async_copy(
            idx_hbm.at[pl.ds(base + par * _K, _K)], idx_bufs[par], sin[par]
        ).start()
    tcp.wait()

    def _gather_chunk(par):
        def vec(j, c2):
            base_w = j * 128
            ids = [
                idx_bufs[par][pl.ds(base_w + k * 16, 16)] for k in range(8)
            ]
            vals = [plsc.load_gather(tbl_v, [iv]) for iv in ids]
            for k in range(8):
                out_bufs[par][pl.ds(base_w + k * 16, 16)] = vals[k]
            return c2

        lax.fori_loop(0, _K // 128, vec, 0)

    for par in range(2):
        off = base + par * _K
        pltpu.make_async_copy(
            idx_hbm.at[pl.ds(off, _K)], idx_bufs[par], sin[par]
        ).wait()
        _gather_chunk(par)
        pltpu.make_async_copy(
            out_bufs[par], out_hbm.at[pl.ds(off, _K)], sout[par]
        ).start()
        pltpu.make_async_copy(
            idx_hbm.at[pl.ds(off + 2 * _K, _K)], idx_bufs[par], sin[par]
        ).start()

    def pair(p, carry):
        for par in range(2):
            ci = p * 2 + par
            off = base + ci * _K
            pltpu.make_async_copy(
                idx_hbm.at[pl.ds(off, _K)], idx_bufs[par], sin[par]
            ).wait()
            pltpu.make_async_copy(
                out_bufs[par], out_hbm.at[pl.ds(off - 2 * _K, _K)], sout[par]
            ).wait()
            _gather_chunk(par)
            pltpu.make_async_copy(
                out_bufs[par], out_hbm.at[pl.ds(off, _K)], sout[par]
            ).start()

            @pl.when(ci + 2 < _NCHUNK)
            def _():
                pltpu.make_async_copy(
                    idx_hbm.at[pl.ds(off + 2 * _K, _K)], idx_bufs[par], sin[par]
                ).start()

        return carry

    lax.fori_loop(1, _NCHUNK // 2, pair, 0)

    for par in range(2):
        off = base + (_NCHUNK - 2 + par) * _K
        pltpu.make_async_copy(
            out_bufs[par], out_hbm.at[pl.ds(off, _K)], sout[par]
        ).wait()


_sc_gather = functools.partial(
    pl.kernel,
    mesh=plsc.VectorSubcoreMesh(core_axis_name="c", subcore_axis_name="s"),
    compiler_params=pltpu.CompilerParams(needs_layout_passes=False),
    out_type=jax.ShapeDtypeStruct((_N,), jnp.float32),
    scratch_types=[
        pltpu.VMEM((_NRELS,), jnp.float32),
        pltpu.VMEM((_K,), jnp.int32),
        pltpu.VMEM((_K,), jnp.int32),
        pltpu.VMEM((_K,), jnp.float32),
        pltpu.VMEM((_K,), jnp.float32),
        pltpu.SemaphoreType.DMA,
        pltpu.SemaphoreType.DMA,
        pltpu.SemaphoreType.DMA,
        pltpu.SemaphoreType.DMA,
        pltpu.SemaphoreType.DMA,
    ],
)(_sc_gather_body)


_W = 1024


def _tc_body(c_ref, x_ref, o_ref):
    def step(l, acc):
        return acc + x_ref[l] * c_ref[l][None, :]

    t = lax.fori_loop(
        0, _L, step, jnp.zeros((_C, _W), jnp.float32), unroll=8
    )
    m = jnp.max(t, axis=0, keepdims=True)
    e = jnp.exp(t - m)
    o_ref[...] = e / jnp.sum(e, axis=0, keepdims=True)


def _tile_flat(a2d):
    return a2d.reshape(_L // 8, 8, _B // 128, 128).transpose(0, 2, 1, 3).reshape(_N)


def _tile_unflat(flat):
    return (
        flat.reshape(_L // 8, _B // 128, 8, 128)
        .transpose(0, 2, 1, 3)
        .reshape(_L, _B)
    )


def kernel(rel_indices, x, d, b):
    del b
    xT = jnp.transpose(x, (1, 2, 0))
    relT = jnp.transpose(rel_indices, (1, 0))
    idx_flat = _tile_flat(relT)
    tbl = d.reshape(_NRELS)
    cT = _tile_unflat(_sc_gather(tbl, idx_flat))
    outT = pl.pallas_call(
        _tc_body,
        grid=(_B // _W,),
        in_specs=[
            pl.BlockSpec((_L, _W), lambda i: (0, i)),
            pl.BlockSpec((_L, _C, _W), lambda i: (0, 0, i)),
        ],
        out_specs=pl.BlockSpec((_C, _W), lambda i: (0, i)),
        out_shape=jax.ShapeDtypeStruct((_C, _B), jnp.float32),
    )(cT, xT)
    return jnp.transpose(outT, (1, 0))


# speedup vs baseline: 1.0837x; 1.0837x over previous
import functools

import jax
import jax.numpy as jnp
from jax import lax
from jax.experimental import pallas as pl
from jax.experimental.pallas import tpu as pltpu
from jax.experimental.pallas import tpu_sc as plsc

_NRELS = 100000
_B = 16384
_L = 200
_C = 16

_NC = 2
_NS = 16
_NW = _NC * _NS
_N = _B * _L
_NPW = _N // _NW
_K = 5120
_NCHUNK = _NPW // _K


def _sc_gather_body(
    tbl_hbm, idx_hbm, out_hbm, tbl_v,
    idx_v0, idx_v1, out_v0, out_v1,
    sem_t, si0, si1, so0, so1,
):
    wid = lax.axis_index("s") * _NC + lax.axis_index("c")
    base = wid * _NPW
    idx_bufs = (idx_v0, idx_v1)
    out_bufs = (out_v0, out_v1)
    sin = (si0, si1)
    sout = (so0, so1)

    tcp = pltpu.make_async_copy(tbl_hbm, tbl_v, sem_t)
    tcp.start()
    for par in range(2):
        pltpu.make_async_copy(
            idx_hbm.at[pl.ds(base + par * _K, _K)], idx_bufs[par], sin[par]
        ).start()
    tcp.wait()

    def _gather_chunk(par):
        def vec(j, c2):
            base_w = j * 128
            ids = [
                idx_bufs[par][pl.ds(base_w + k * 16, 16)] for k in range(8)
            ]
            vals = [plsc.load_gather(tbl_v, [iv]) for iv in ids]
            for k in range(8):
                out_bufs[par][pl.ds(base_w + k * 16, 16)] = vals[k]
            return c2

        lax.fori_loop(0, _K // 128, vec, 0)

    for par in range(2):
        off = base + par * _K
        pltpu.make_async_copy(
            idx_hbm.at[pl.ds(off, _K)], idx_bufs[par], sin[par]
        ).wait()
        _gather_chunk(par)
        pltpu.make_async_copy(
            out_bufs[par], out_hbm.at[pl.ds(off, _K)], sout[par]
        ).start()
        pltpu.make_async_copy(
            idx_hbm.at[pl.ds(off + 2 * _K, _K)], idx_bufs[par], sin[par]
        ).start()

    def pair(p, carry):
        for par in range(2):
            ci = p * 2 + par
            off = base + ci * _K
            pltpu.make_async_copy(
                idx_hbm.at[pl.ds(off, _K)], idx_bufs[par], sin[par]
            ).wait()
            pltpu.make_async_copy(
                out_bufs[par], out_hbm.at[pl.ds(off - 2 * _K, _K)], sout[par]
            ).wait()
            _gather_chunk(par)
            pltpu.make_async_copy(
                out_bufs[par], out_hbm.at[pl.ds(off, _K)], sout[par]
            ).start()

            @pl.when(ci + 2 < _NCHUNK)
            def _():
                pltpu.make_async_copy(
                    idx_hbm.at[pl.ds(off + 2 * _K, _K)], idx_bufs[par], sin[par]
                ).start()

        return carry

    lax.fori_loop(1, _NCHUNK // 2, pair, 0)

    for par in range(2):
        off = base + (_NCHUNK - 2 + par) * _K
        pltpu.make_async_copy(
            out_bufs[par], out_hbm.at[pl.ds(off, _K)], sout[par]
        ).wait()


_sc_gather = functools.partial(
    pl.kernel,
    mesh=plsc.VectorSubcoreMesh(core_axis_name="c", subcore_axis_name="s"),
    compiler_params=pltpu.CompilerParams(needs_layout_passes=False),
    out_type=jax.ShapeDtypeStruct((_N,), jnp.float32),
    scratch_types=[
        pltpu.VMEM((_NRELS,), jnp.float32),
        pltpu.VMEM((_K,), jnp.int32),
        pltpu.VMEM((_K,), jnp.int32),
        pltpu.VMEM((_K,), jnp.float32),
        pltpu.VMEM((_K,), jnp.float32),
        pltpu.SemaphoreType.DMA,
        pltpu.SemaphoreType.DMA,
        pltpu.SemaphoreType.DMA,
        pltpu.SemaphoreType.DMA,
        pltpu.SemaphoreType.DMA,
    ],
)(_sc_gather_body)


_W = 1024


def _tc_body(c_ref, x_ref, o_ref):
    def step(l, acc):
        return acc + x_ref[l] * c_ref[l][None, :]

    t = lax.fori_loop(
        0, _L, step, jnp.zeros((_C, _W), jnp.float32), unroll=8
    )
    m = jnp.max(t, axis=0, keepdims=True)
    e = jnp.exp(t - m)
    o_ref[...] = e / jnp.sum(e, axis=0, keepdims=True)


def _tile_flat(a2d):
    return a2d.reshape(_L // 8, 8, _B // 128, 128).transpose(0, 2, 1, 3).reshape(_N)


def _tile_unflat(flat):
    return (
        flat.reshape(_L // 8, _B // 128, 8, 128)
        .transpose(0, 2, 1, 3)
        .reshape(_L, _B)
    )


def kernel(rel_indices, x, d, b):
    del b
    xT = jnp.transpose(x, (1, 2, 0))
    relT = jnp.transpose(rel_indices, (1, 0))
    idx_flat = _tile_flat(relT)
    tbl = d.reshape(_NRELS)
    cT = _tile_unflat(_sc_gather(tbl, idx_flat))
    outT = pl.pallas_call(
        _tc_body,
        grid=(_B // _W,),
        in_specs=[
            pl.BlockSpec((_L, _W), lambda i: (0, i)),
            pl.BlockSpec((_L, _C, _W), lambda i: (0, 0, i)),
        ],
        out_specs=pl.BlockSpec((_C, _W), lambda i: (0, i)),
        out_shape=jax.ShapeDtypeStruct((_C, _B), jnp.float32),
    )(cT, xT)
    return jnp.transpose(outT, (1, 0))
